# baseline (speedup 1.0000x reference)
import jax
import jax.numpy as jnp
from jax import lax
from jax.experimental import pallas as pl
from jax.experimental.pallas import tpu as pltpu
from jax.experimental.pallas import tpu_sc as plsc

_NC, _NS = 2, 16
_NW = _NC * _NS
_TM = 512
_D = 128


def _ru(x, m):
    return (x + m - 1) // m * m


def _pick_chunk(b_per_w):
    best = 8
    e = 8
    while e <= min(b_per_w, 768):
        if b_per_w % e == 0:
            best = e
        e += 8
    return best


def _sc_gather(table, idx):
    v, d = table.shape
    (b,) = idx.shape
    b_per_w = b // _NW
    cb = _pick_chunk(b_per_w)
    nch = b_per_w // cb
    mesh = plsc.VectorSubcoreMesh(core_axis_name="c", subcore_axis_name="s")

    def body(table_hbm, idx_hbm, out_hbm, idx_v, rows_v, sem):
        wid = lax.axis_index("s") * _NC + lax.axis_index("c")
        base = wid * b_per_w

        def step(c, carry):
            off = base + c * cb
            pltpu.sync_copy(idx_hbm.at[pl.ds(off, cb)], idx_v)
            pltpu.async_copy(table_hbm.at[idx_v], rows_v, sem).wait()
            pltpu.sync_copy(rows_v, out_hbm.at[pl.ds(off, cb)])
            return carry

        lax.fori_loop(0, nch, step, 0)

    fn = pl.kernel(
        body,
        out_type=jax.ShapeDtypeStruct((b, d), jnp.float32),
        mesh=mesh,
        scratch_types=[
            pltpu.VMEM((cb,), jnp.int32),
            pltpu.VMEM((cb, d), jnp.float32),
            pltpu.SemaphoreType.DMA,
        ],
    )
    return fn(table, idx)


def _sconv_gemm(g, wk, bias, m_real, cin, relu=False, x=None, xw=None,
                n_out=_D):
    m_pad = g.shape[0]
    taps, _, cout = wk.shape
    width = xw if x is not None else cout

    def body(*refs):
        if x is not None:
            a_ref, w_ref, b_ref, x_ref, o_ref, acc_ref = refs
        else:
            a_ref, w_ref, b_ref, o_ref, acc_ref = refs
        i = pl.program_id(0)
        k = pl.program_id(1)

        @pl.when(k == 0)
        def _():
            acc_ref[...] = jnp.zeros_like(acc_ref)

        acc_ref[...] += jnp.dot(
            a_ref[:, :cin], w_ref[0], preferred_element_type=jnp.float32
        )

        @pl.when(k == taps - 1)
        def _():
            acc = acc_ref[...] + b_ref[...]
            if relu:
                acc = jnp.maximum(acc, 0.0)
            if x is not None:
                acc = x_ref[:, :xw] * acc[:, :xw] + acc[:, xw:]
            if n_out > width:
                acc = jnp.pad(acc, ((0, 0), (0, n_out - width)))
            rows = lax.broadcasted_iota(jnp.int32, (_TM, n_out), 0) + i * _TM
            o_ref[...] = jnp.where(rows < m_real, acc, 0.0)

    in_specs = [
        pl.BlockSpec((_TM, _D), lambda i, k: (i, k)),
        pl.BlockSpec((1, cin, cout), lambda i, k: (k, 0, 0)),
        pl.BlockSpec((1, cout), lambda i, k: (0, 0)),
    ]
    args = [g, wk[:, :cin, :], bias.reshape(1, cout)]
    if x is not None:
        in_specs.append(pl.BlockSpec((_TM, _D), lambda i, k: (i, 0)))
        args.append(x)
    return pl.pallas_call(
        body,
        grid=(m_pad // _TM, taps),
        in_specs=in_specs,
        out_specs=pl.BlockSpec((_TM, n_out), lambda i, k: (i, 0)),
        out_shape=jax.ShapeDtypeStruct((m_pad, n_out), jnp.float32),
        scratch_shapes=[pltpu.VMEM((_TM, cout), jnp.float32)],
    )(*args)


def _up_gemm(feat, wk, bias, m_real, cin):
    m_pad = feat.shape[0]
    _, _, cout = wk.shape
    w = jnp.zeros((cin, 8, _D), jnp.float32)
    w = w.at[:, :, :cout].set(wk[:, :cin, :].transpose(1, 0, 2))
    w = w.reshape(cin, 8 * _D)
    b = jnp.zeros((8, _D), jnp.float32).at[:, :cout].set(bias).reshape(1, -1)

    def body(a_ref, w_ref, b_ref, o_ref):
        i = pl.program_id(0)
        acc = jnp.dot(a_ref[:, :cin], w_ref[...],
                      preferred_element_type=jnp.float32)
        acc = jnp.maximum(acc + b_ref[...], 0.0)
        rows = lax.broadcasted_iota(jnp.int32, (_TM, 8 * _D), 0) + i * _TM
        o_ref[...] = jnp.where(rows < m_real, acc, 0.0)

    return pl.pallas_call(
        body,
        grid=(m_pad // _TM,),
        in_specs=[
            pl.BlockSpec((_TM, _D), lambda i: (i, 0)),
            pl.BlockSpec((cin, 8 * _D), lambda i: (0, 0)),
            pl.BlockSpec((1, 8 * _D), lambda i: (0, 0)),
        ],
        out_specs=pl.BlockSpec((_TM, 8 * _D), lambda i: (i, 0)),
        out_shape=jax.ShapeDtypeStruct((m_pad, 8 * _D), jnp.float32),
    )(feat, w, b)


def _neighbor_table(km, m, m_pad):
    oo = jnp.concatenate([km[k][1] for k in range(27)])
    ii = jnp.concatenate([km[k][0] for k in range(27)])
    kk = jnp.concatenate(
        [jnp.full(km[k][0].shape, k, jnp.int32) for k in range(27)]
    )
    tab = jnp.full((m_pad * 27,), m, jnp.int32)
    return tab.at[oo * 27 + kk].set(
        ii, unique_indices=True, mode="promise_in_bounds"
    )


def _up_index(um, m_out_pad, m_in):
    fi = jnp.concatenate([um[k][0] for k in range(8)])
    pk = jnp.concatenate([um[k][1] * 8 + k for k in range(8)])
    idx = jnp.full((m_out_pad,), m_in * 8, jnp.int32)
    return idx.at[fi].set(
        pk.astype(jnp.int32), unique_indices=True, mode="promise_in_bounds"
    )


def _sconv(feat, cin, wk, bias, ntab, m, relu=False, x=None, xw=None,
           n_out=_D):
    g = _sc_gather(feat, ntab)
    a = g.reshape(-1, 27 * _D)
    return _sconv_gemm(a, wk, bias, m, cin, relu=relu, x=x, xw=xw,
                       n_out=n_out)


def _upconv(feat, cin, wk, bias, idx, m_in):
    p = _up_gemm(feat, wk, bias, m_in, cin)
    return _sc_gather(p.reshape(-1, _D), idx)


def kernel(y_feat, params, km8, km4, km2, km1, up84, up42, up21, M8, M4, M2, M1):
    p = params
    m8 = y_feat.shape[0]
    m4 = km4[13][0].shape[0]
    m2 = km2[13][0].shape[0]
    m1 = km1[13][0].shape[0]
    p8, p4, p2, p1 = (_ru(m + 1, _TM) for m in (m8, m4, m2, m1))

    n8 = _neighbor_table(km8, m8, p8)
    n4 = _neighbor_table(km4, m4, p4)
    n2 = _neighbor_table(km2, m2, p2)
    n1 = _neighbor_table(km1, m1, p1)
    i84 = _up_index(up84, p4, m8)
    i42 = _up_index(up42, p2, m4)
    i21 = _up_index(up21, p1, m2)

    x = jnp.pad(y_feat[:, :128], ((0, p8 - m8), (0, 0)))
    q = jnp.pad(y_feat[:, 128:], ((0, p8 - m8), (0, _D - 32)))

    x = _sconv(x, 128, p['pre_w'], p['pre_b'], n8, m8, relu=True)
    q = _sconv(q, 32, p['qpre_w'], p['qpre_b'], n8, m8, relu=True)
    q = _sconv(q, 32, p['ql1_w'], p['ql1_b'], n8, m8, relu=True)
    h = _sconv(q, 32, p['qp1_w1'], p['qp1_b1'], n8, m8, relu=True)
    x = _sconv(h, 128, p['qp1_w2'], p['qp1_b2'], n8, m8, x=x, xw=128)

    q = _upconv(q, 32, p['qup1_w'], p['qup1_b'], i84, m8)
    x = _upconv(x, 128, p['up1_w'], p['up1_b'], i84, m8)

    q = _sconv(q, 32, p['ql2_w'], p['ql2_b'], n4, m4, relu=True)
    h = _sconv(q, 32, p['qp2_w1'], p['qp2_b1'], n4, m4, relu=True)
    x = _sconv(h, 128, p['qp2_w2'], p['qp2_b2'], n4, m4, x=x, xw=128)

    q = _upconv(q, 32, p['qup2_w'], p['qup2_b'], i42, m4)
    x = _upconv(x, 128, p['up2_w'], p['up2_b'], i42, m4)

    q = _sconv(q, 16, p['ql3_w'], p['ql3_b'], n2, m2)
    h = _sconv(q, 16, p['qp3_w1'], p['qp3_b1'], n2, m2, relu=True)
    x = _sconv(h, 64, p['qp3_w2'], p['qp3_b2'], n2, m2, x=x, xw=64)

    x = _upconv(x, 64, p['up3_w'], p['up3_b'], i21, m2)
    x = _sconv(x, 32, p['post_w'], p['post_b'], n1, m1, n_out=3)
    return x[:m1]


# speedup vs baseline: 1.0011x; 1.0011x over previous
import jax
import jax.numpy as jnp
from jax import lax
from jax.experimental import pallas as pl
from jax.experimental.pallas import tpu as pltpu
from jax.experimental.pallas import tpu_sc as plsc

_NC, _NS = 2, 16
_NW = _NC * _NS
_TM = 512
_D = 128


def _ru(x, m):
    return (x + m - 1) // m * m


_NBUF = 4


def _pick_chunk(b_per_w, cap):
    best = 8
    e = 8
    while e <= min(b_per_w, cap):
        if b_per_w % e == 0:
            best = e
        e += 8
    return best


def _sc_gather(table, idx):
    v, d = table.shape
    (b,) = idx.shape
    b_per_w = b // _NW
    cap = 124000 // (_NBUF * (d + 1))
    cb = _pick_chunk(b_per_w, cap)
    nch = b_per_w // cb
    nround = -(-nch // _NBUF)
    mesh = plsc.VectorSubcoreMesh(core_axis_name="c", subcore_axis_name="s")

    def body(table_hbm, idx_hbm, out_hbm, *scratch):
        idx_v = scratch[0:_NBUF]
        rows_v = scratch[_NBUF:2 * _NBUF]
        sem_i = scratch[2 * _NBUF:3 * _NBUF]
        sem_g = scratch[3 * _NBUF:4 * _NBUF]
        sem_o = scratch[4 * _NBUF:5 * _NBUF]
        wid = lax.axis_index("s") * _NC + lax.axis_index("c")
        base = wid * b_per_w

        def idx_copy(c, bf):
            return pltpu.make_async_copy(
                idx_hbm.at[pl.ds(base + c * cb, cb)], idx_v[bf], sem_i[bf])

        def out_copy(c, bf):
            return pltpu.make_async_copy(
                rows_v[bf], out_hbm.at[pl.ds(base + c * cb, cb)], sem_o[bf])

        def gat_copy(bf):
            return pltpu.make_async_copy(
                table_hbm.at[idx_v[bf]], rows_v[bf], sem_g[bf])

        for bf in range(min(_NBUF, nch)):
            idx_copy(bf, bf).start()

        def round_fn(r, carry):
            for bf in range(_NBUF):
                c = r * _NBUF + bf

                @pl.when(c < nch)
                def _():
                    @pl.when(r > 0)
                    def _():
                        out_copy(c, bf).wait()
                    idx_copy(c, bf).wait()
                    gat_copy(bf).start()

            for bf in range(_NBUF):
                c = r * _NBUF + bf

                @pl.when(c < nch)
                def _():
                    gat_copy(bf).wait()
                    out_copy(c, bf).start()

                    @pl.when(c + _NBUF < nch)
                    def _():
                        idx_copy(c + _NBUF, bf).start()

            return carry

        lax.fori_loop(0, nround, round_fn, 0)
        for bf in range(min(_NBUF, nch)):
            out_copy(0, bf).wait()

    fn = pl.kernel(
        body,
        out_type=jax.ShapeDtypeStruct((b, d), jnp.float32),
        mesh=mesh,
        scratch_types=(
            [pltpu.VMEM((cb,), jnp.int32) for _ in range(_NBUF)]
            + [pltpu.VMEM((cb, d), jnp.float32) for _ in range(_NBUF)]
            + [pltpu.SemaphoreType.DMA for _ in range(3 * _NBUF)]
        ),
    )
    return fn(table, idx)


def _sconv_gemm(g, wk, bias, m_real, cin, relu=False, x=None, xw=None,
                n_out=_D):
    m_pad = g.shape[0]
    taps, _, cout = wk.shape
    width = xw if x is not None else cout

    def body(*refs):
        if x is not None:
            a_ref, w_ref, b_ref, x_ref, o_ref, acc_ref = refs
        else:
            a_ref, w_ref, b_ref, o_ref, acc_ref = refs
        i = pl.program_id(0)
        k = pl.program_id(1)

        @pl.when(k == 0)
        def _():
            acc_ref[...] = jnp.zeros_like(acc_ref)

        acc_ref[...] += jnp.dot(
            a_ref[:, :cin], w_ref[0], preferred_element_type=jnp.float32
        )

        @pl.when(k == taps - 1)
        def _():
            acc = acc_ref[...] + b_ref[...]
            if relu:
                acc = jnp.maximum(acc, 0.0)
            if x is not None:
                acc = x_ref[:, :xw] * acc[:, :xw] + acc[:, xw:]
            if n_out > width:
                acc = jnp.pad(acc, ((0, 0), (0, n_out - width)))
            rows = lax.broadcasted_iota(jnp.int32, (_TM, n_out), 0) + i * _TM
            o_ref[...] = jnp.where(rows < m_real, acc, 0.0)

    in_specs = [
        pl.BlockSpec((_TM, _D), lambda i, k: (i, k)),
        pl.BlockSpec((1, cin, cout), lambda i, k: (k, 0, 0)),
        pl.BlockSpec((1, cout), lambda i, k: (0, 0)),
    ]
    args = [g, wk[:, :cin, :], bias.reshape(1, cout)]
    if x is not None:
        in_specs.append(pl.BlockSpec((_TM, _D), lambda i, k: (i, 0)))
        args.append(x)
    return pl.pallas_call(
        body,
        grid=(m_pad // _TM, taps),
        in_specs=in_specs,
        out_specs=pl.BlockSpec((_TM, n_out), lambda i, k: (i, 0)),
        out_shape=jax.ShapeDtypeStruct((m_pad, n_out), jnp.float32),
        scratch_shapes=[pltpu.VMEM((_TM, cout), jnp.float32)],
    )(*args)


def _up_gemm(feat, wk, bias, m_real, cin):
    m_pad = feat.shape[0]
    _, _, cout = wk.shape
    w = jnp.zeros((cin, 8, _D), jnp.float32)
    w = w.at[:, :, :cout].set(wk[:, :cin, :].transpose(1, 0, 2))
    w = w.reshape(cin, 8 * _D)
    b = jnp.zeros((8, _D), jnp.float32).at[:, :cout].set(bias).reshape(1, -1)

    def body(a_ref, w_ref, b_ref, o_ref):
        i = pl.program_id(0)
        acc = jnp.dot(a_ref[:, :cin], w_ref[...],
                      preferred_element_type=jnp.float32)
        acc = jnp.maximum(acc + b_ref[...], 0.0)
        rows = lax.broadcasted_iota(jnp.int32, (_TM, 8 * _D), 0) + i * _TM
        o_ref[...] = jnp.where(rows < m_real, acc, 0.0)

    return pl.pallas_call(
        body,
        grid=(m_pad // _TM,),
        in_specs=[
            pl.BlockSpec((_TM, _D), lambda i: (i, 0)),
            pl.BlockSpec((cin, 8 * _D), lambda i: (0, 0)),
            pl.BlockSpec((1, 8 * _D), lambda i: (0, 0)),
        ],
        out_specs=pl.BlockSpec((_TM, 8 * _D), lambda i: (i, 0)),
        out_shape=jax.ShapeDtypeStruct((m_pad, 8 * _D), jnp.float32),
    )(feat, w, b)


def _neighbor_table(km, m, m_pad):
    oo = jnp.concatenate([km[k][1] for k in range(27)])
    ii = jnp.concatenate([km[k][0] for k in range(27)])
    kk = jnp.concatenate(
        [jnp.full(km[k][0].shape, k, jnp.int32) for k in range(27)]
    )
    tab = jnp.full((m_pad * 27,), m, jnp.int32)
    return tab.at[oo * 27 + kk].set(
        ii, unique_indices=True, mode="promise_in_bounds"
    )


def _up_index(um, m_out_pad, m_in):
    fi = jnp.concatenate([um[k][0] for k in range(8)])
    pk = jnp.concatenate([um[k][1] * 8 + k for k in range(8)])
    idx = jnp.full((m_out_pad,), m_in * 8, jnp.int32)
    return idx.at[fi].set(
        pk.astype(jnp.int32), unique_indices=True, mode="promise_in_bounds"
    )


def _sconv(feat, cin, wk, bias, ntab, m, relu=False, x=None, xw=None,
           n_out=_D):
    g = _sc_gather(feat, ntab)
    a = g.reshape(-1, 27 * _D)
    return _sconv_gemm(a, wk, bias, m, cin, relu=relu, x=x, xw=xw,
                       n_out=n_out)


def _upconv(feat, cin, wk, bias, idx, m_in):
    p = _up_gemm(feat, wk, bias, m_in, cin)
    return _sc_gather(p.reshape(-1, _D), idx)


def kernel(y_feat, params, km8, km4, km2, km1, up84, up42, up21, M8, M4, M2, M1):
    p = params
    m8 = y_feat.shape[0]
    m4 = km4[13][0].shape[0]
    m2 = km2[13][0].shape[0]
    m1 = km1[13][0].shape[0]
    p8, p4, p2, p1 = (_ru(m + 1, _TM) for m in (m8, m4, m2, m1))

    n8 = _neighbor_table(km8, m8, p8)
    n4 = _neighbor_table(km4, m4, p4)
    n2 = _neighbor_table(km2, m2, p2)
    n1 = _neighbor_table(km1, m1, p1)
    i84 = _up_index(up84, p4, m8)
    i42 = _up_index(up42, p2, m4)
    i21 = _up_index(up21, p1, m2)

    x = jnp.pad(y_feat[:, :128], ((0, p8 - m8), (0, 0)))
    q = jnp.pad(y_feat[:, 128:], ((0, p8 - m8), (0, _D - 32)))

    x = _sconv(x, 128, p['pre_w'], p['pre_b'], n8, m8, relu=True)
    q = _sconv(q, 32, p['qpre_w'], p['qpre_b'], n8, m8, relu=True)
    q = _sconv(q, 32, p['ql1_w'], p['ql1_b'], n8, m8, relu=True)
    h = _sconv(q, 32, p['qp1_w1'], p['qp1_b1'], n8, m8, relu=True)
    x = _sconv(h, 128, p['qp1_w2'], p['qp1_b2'], n8, m8, x=x, xw=128)

    q = _upconv(q, 32, p['qup1_w'], p['qup1_b'], i84, m8)
    x = _upconv(x, 128, p['up1_w'], p['up1_b'], i84, m8)

    q = _sconv(q, 32, p['ql2_w'], p['ql2_b'], n4, m4, relu=True)
    h = _sconv(q, 32, p['qp2_w1'], p['qp2_b1'], n4, m4, relu=True)
    x = _sconv(h, 128, p['qp2_w2'], p['qp2_b2'], n4, m4, x=x, xw=128)

    q = _upconv(q, 32, p['qup2_w'], p['qup2_b'], i42, m4)
    x = _upconv(x, 128, p['up2_w'], p['up2_b'], i42, m4)

    q = _sconv(q, 16, p['ql3_w'], p['ql3_b'], n2, m2)
    h = _sconv(q, 16, p['qp3_w1'], p['qp3_b1'], n2, m2, relu=True)
    x = _sconv(h, 64, p['qp3_w2'], p['qp3_b2'], n2, m2, x=x, xw=64)

    x = _upconv(x, 64, p['up3_w'], p['up3_b'], i21, m2)
    x = _sconv(x, 32, p['post_w'], p['post_b'], n1, m1, n_out=3)
    return x[:m1]


# speedup vs baseline: 4.5562x; 4.5513x over previous
import jax
import jax.numpy as jnp
from jax import lax
from jax.experimental import pallas as pl
from jax.experimental.pallas import tpu as pltpu
from jax.experimental.pallas import tpu_sc as plsc

_NC, _NS = 2, 16
_NW = _NC * _NS
_TM = 512
_D = 128


def _ru(x, m):
    return (x + m - 1) // m * m


_NBUF = 4


def _pick_chunk(b_per_w, cap):
    best = 8
    e = 8
    while e <= min(b_per_w, cap):
        if b_per_w % e == 0:
            best = e
        e += 8
    return best


def _sc_gather(table, idx):
    v, d = table.shape
    (b,) = idx.shape
    b_per_w = b // _NW
    cap = 124000 // (_NBUF * (d + 1))
    cb = _pick_chunk(b_per_w, cap)
    nch = b_per_w // cb
    nround = -(-nch // _NBUF)
    mesh = plsc.VectorSubcoreMesh(core_axis_name="c", subcore_axis_name="s")

    def body(table_hbm, idx_hbm, out_hbm, *scratch):
        idx_v = scratch[0:_NBUF]
        rows_v = scratch[_NBUF:2 * _NBUF]
        sem_i = scratch[2 * _NBUF:3 * _NBUF]
        sem_g = scratch[3 * _NBUF:4 * _NBUF]
        sem_o = scratch[4 * _NBUF:5 * _NBUF]
        wid = lax.axis_index("s") * _NC + lax.axis_index("c")
        base = wid * b_per_w

        def idx_copy(c, bf):
            return pltpu.make_async_copy(
                idx_hbm.at[pl.ds(base + c * cb, cb)], idx_v[bf], sem_i[bf])

        def out_copy(c, bf):
            return pltpu.make_async_copy(
                rows_v[bf], out_hbm.at[pl.ds(base + c * cb, cb)], sem_o[bf])

        def gat_copy(bf):
            return pltpu.make_async_copy(
                table_hbm.at[idx_v[bf]], rows_v[bf], sem_g[bf])

        for bf in range(min(_NBUF, nch)):
            idx_copy(bf, bf).start()

        def round_fn(r, carry):
            for bf in range(_NBUF):
                c = r * _NBUF + bf

                @pl.when(c < nch)
                def _():
                    @pl.when(r > 0)
                    def _():
                        out_copy(c, bf).wait()
                    idx_copy(c, bf).wait()
                    gat_copy(bf).start()

            for bf in range(_NBUF):
                c = r * _NBUF + bf

                @pl.when(c < nch)
                def _():
                    gat_copy(bf).wait()
                    out_copy(c, bf).start()

                    @pl.when(c + _NBUF < nch)
                    def _():
                        idx_copy(c + _NBUF, bf).start()

            return carry

        lax.fori_loop(0, nround, round_fn, 0)
        for bf in range(min(_NBUF, nch)):
            out_copy(0, bf).wait()

    fn = pl.kernel(
        body,
        out_type=jax.ShapeDtypeStruct((b, d), jnp.float32),
        mesh=mesh,
        scratch_types=(
            [pltpu.VMEM((cb,), jnp.int32) for _ in range(_NBUF)]
            + [pltpu.VMEM((cb, d), jnp.float32) for _ in range(_NBUF)]
            + [pltpu.SemaphoreType.DMA for _ in range(3 * _NBUF)]
        ),
    )
    return fn(table, idx)


def _sconv_gemm(g, wk, bias, m_real, cin, relu=False, x=None, xw=None,
                n_out=_D):
    m_pad = g.shape[0]
    taps, _, cout = wk.shape
    width = xw if x is not None else cout

    def body(*refs):
        if x is not None:
            a_ref, w_ref, b_ref, x_ref, o_ref, acc_ref = refs
        else:
            a_ref, w_ref, b_ref, o_ref, acc_ref = refs
        i = pl.program_id(0)
        k = pl.program_id(1)

        @pl.when(k == 0)
        def _():
            acc_ref[...] = jnp.zeros_like(acc_ref)

        acc_ref[...] += jnp.dot(
            a_ref[:, :cin], w_ref[0], preferred_element_type=jnp.float32
        )

        @pl.when(k == taps - 1)
        def _():
            acc = acc_ref[...] + b_ref[...]
            if relu:
                acc = jnp.maximum(acc, 0.0)
            if x is not None:
                acc = x_ref[:, :xw] * acc[:, :xw] + acc[:, xw:]
            if n_out > width:
                acc = jnp.pad(acc, ((0, 0), (0, n_out - width)))
            rows = lax.broadcasted_iota(jnp.int32, (_TM, n_out), 0) + i * _TM
            o_ref[...] = jnp.where(rows < m_real, acc, 0.0)

    in_specs = [
        pl.BlockSpec((_TM, _D), lambda i, k: (i, k)),
        pl.BlockSpec((1, cin, cout), lambda i, k: (k, 0, 0)),
        pl.BlockSpec((1, cout), lambda i, k: (0, 0)),
    ]
    args = [g, wk[:, :cin, :], bias.reshape(1, cout)]
    if x is not None:
        in_specs.append(pl.BlockSpec((_TM, _D), lambda i, k: (i, 0)))
        args.append(x)
    return pl.pallas_call(
        body,
        grid=(m_pad // _TM, taps),
        in_specs=in_specs,
        out_specs=pl.BlockSpec((_TM, n_out), lambda i, k: (i, 0)),
        out_shape=jax.ShapeDtypeStruct((m_pad, n_out), jnp.float32),
        scratch_shapes=[pltpu.VMEM((_TM, cout), jnp.float32)],
    )(*args)


def _up_gemm(feat, wk, bias, m_real, cin):
    m_pad = feat.shape[0]
    _, _, cout = wk.shape
    w = jnp.zeros((cin, 8, _D), jnp.float32)
    w = w.at[:, :, :cout].set(wk[:, :cin, :].transpose(1, 0, 2))
    w = w.reshape(cin, 8 * _D)
    b = jnp.zeros((8, _D), jnp.float32).at[:, :cout].set(bias).reshape(1, -1)

    def body(a_ref, w_ref, b_ref, o_ref):
        i = pl.program_id(0)
        acc = jnp.dot(a_ref[:, :cin], w_ref[...],
                      preferred_element_type=jnp.float32)
        acc = jnp.maximum(acc + b_ref[...], 0.0)
        rows = lax.broadcasted_iota(jnp.int32, (_TM, 8 * _D), 0) + i * _TM
        o_ref[...] = jnp.where(rows < m_real, acc, 0.0)

    return pl.pallas_call(
        body,
        grid=(m_pad // _TM,),
        in_specs=[
            pl.BlockSpec((_TM, _D), lambda i: (i, 0)),
            pl.BlockSpec((cin, 8 * _D), lambda i: (0, 0)),
            pl.BlockSpec((1, 8 * _D), lambda i: (0, 0)),
        ],
        out_specs=pl.BlockSpec((_TM, 8 * _D), lambda i: (i, 0)),
        out_shape=jax.ShapeDtypeStruct((m_pad, 8 * _D), jnp.float32),
    )(feat, w, b)


def _neighbor_table(km, m, m_pad):
    oo = jnp.concatenate([km[k][1] for k in range(27)])
    ii = jnp.concatenate([km[k][0] for k in range(27)])
    kk = jnp.concatenate(
        [jnp.full(km[k][0].shape, k, jnp.int32) for k in range(27)]
    )
    tab = m + jnp.arange(m_pad * 27, dtype=jnp.int32) % (m_pad - m)
    return tab.at[oo * 27 + kk].set(
        ii, unique_indices=True, mode="promise_in_bounds"
    )


def _up_index(um, m_out_pad, m_in, m_in_pad):
    fi = jnp.concatenate([um[k][0] for k in range(8)])
    pk = jnp.concatenate([um[k][1] * 8 + k for k in range(8)])
    idx = (m_in + jnp.arange(m_out_pad, dtype=jnp.int32)
           % (m_in_pad - m_in)) * 8
    return idx.at[fi].set(
        pk.astype(jnp.int32), unique_indices=True, mode="promise_in_bounds"
    )


def _sconv(feat, cin, wk, bias, ntab, m, relu=False, x=None, xw=None,
           n_out=_D):
    g = _sc_gather(feat, ntab)
    a = g.reshape(-1, 27 * _D)
    return _sconv_gemm(a, wk, bias, m, cin, relu=relu, x=x, xw=xw,
                       n_out=n_out)


def _upconv(feat, cin, wk, bias, idx, m_in):
    p = _up_gemm(feat, wk, bias, m_in, cin)
    return _sc_gather(p.reshape(-1, _D), idx)


def kernel(y_feat, params, km8, km4, km2, km1, up84, up42, up21, M8, M4, M2, M1):
    p = params
    m8 = y_feat.shape[0]
    m4 = km4[13][0].shape[0]
    m2 = km2[13][0].shape[0]
    m1 = km1[13][0].shape[0]
    p8, p4, p2, p1 = (_ru(m + 512, _TM) for m in (m8, m4, m2, m1))

    n8 = _neighbor_table(km8, m8, p8)
    n4 = _neighbor_table(km4, m4, p4)
    n2 = _neighbor_table(km2, m2, p2)
    n1 = _neighbor_table(km1, m1, p1)
    i84 = _up_index(up84, p4, m8, p8)
    i42 = _up_index(up42, p2, m4, p4)
    i21 = _up_index(up21, p1, m2, p2)

    x = jnp.pad(y_feat[:, :128], ((0, p8 - m8), (0, 0)))
    q = jnp.pad(y_feat[:, 128:], ((0, p8 - m8), (0, _D - 32)))

    x = _sconv(x, 128, p['pre_w'], p['pre_b'], n8, m8, relu=True)
    q = _sconv(q, 32, p['qpre_w'], p['qpre_b'], n8, m8, relu=True)
    q = _sconv(q, 32, p['ql1_w'], p['ql1_b'], n8, m8, relu=True)
    h = _sconv(q, 32, p['qp1_w1'], p['qp1_b1'], n8, m8, relu=True)
    x = _sconv(h, 128, p['qp1_w2'], p['qp1_b2'], n8, m8, x=x, xw=128)

    q = _upconv(q, 32, p['qup1_w'], p['qup1_b'], i84, m8)
    x = _upconv(x, 128, p['up1_w'], p['up1_b'], i84, m8)

    q = _sconv(q, 32, p['ql2_w'], p['ql2_b'], n4, m4, relu=True)
    h = _sconv(q, 32, p['qp2_w1'], p['qp2_b1'], n4, m4, relu=True)
    x = _sconv(h, 128, p['qp2_w2'], p['qp2_b2'], n4, m4, x=x, xw=128)

    q = _upconv(q, 32, p['qup2_w'], p['qup2_b'], i42, m4)
    x = _upconv(x, 128, p['up2_w'], p['up2_b'], i42, m4)

    q = _sconv(q, 16, p['ql3_w'], p['ql3_b'], n2, m2)
    h = _sconv(q, 16, p['qp3_w1'], p['qp3_b1'], n2, m2, relu=True)
    x = _sconv(h, 64, p['qp3_w2'], p['qp3_b2'], n2, m2, x=x, xw=64)

    x = _upconv(x, 64, p['up3_w'], p['up3_b'], i21, m2)
    x = _sconv(x, 32, p['post_w'], p['post_b'], n1, m1, n_out=3)
    return x[:m1]


# speedup vs baseline: 4.8762x; 1.0702x over previous
import jax
import jax.numpy as jnp
from jax import lax
from jax.experimental import pallas as pl
from jax.experimental.pallas import tpu as pltpu
from jax.experimental.pallas import tpu_sc as plsc

_NC, _NS = 2, 16
_NW = _NC * _NS
_TM = 512
_D = 128


def _ru(x, m):
    return (x + m - 1) // m * m


_NBUF = 4


def _pick_chunk(b_per_w, cap):
    best = 8
    e = 8
    while e <= min(b_per_w, cap):
        if b_per_w % e == 0:
            best = e
        e += 8
    return best


def _sc_gather(table, idx):
    v, d = table.shape
    (b,) = idx.shape
    b_per_w = b // _NW
    cap = 124000 // (_NBUF * (d + 1))
    cb = _pick_chunk(b_per_w, cap)
    nch = b_per_w // cb
    nround = -(-nch // _NBUF)
    mesh = plsc.VectorSubcoreMesh(core_axis_name="c", subcore_axis_name="s")

    def body(table_hbm, idx_hbm, out_hbm, *scratch):
        idx_v = scratch[0:_NBUF]
        rows_v = scratch[_NBUF:2 * _NBUF]
        sem_i = scratch[2 * _NBUF:3 * _NBUF]
        sem_g = scratch[3 * _NBUF:4 * _NBUF]
        sem_o = scratch[4 * _NBUF:5 * _NBUF]
        wid = lax.axis_index("s") * _NC + lax.axis_index("c")
        base = wid * b_per_w

        def idx_copy(c, bf):
            return pltpu.make_async_copy(
                idx_hbm.at[pl.ds(base + c * cb, cb)], idx_v[bf], sem_i[bf])

        def out_copy(c, bf):
            return pltpu.make_async_copy(
                rows_v[bf], out_hbm.at[pl.ds(base + c * cb, cb)], sem_o[bf])

        def gat_copy(bf):
            return pltpu.make_async_copy(
                table_hbm.at[idx_v[bf]], rows_v[bf], sem_g[bf])

        for bf in range(min(_NBUF, nch)):
            idx_copy(bf, bf).start()

        def round_fn(r, carry):
            for bf in range(_NBUF):
                c = r * _NBUF + bf

                @pl.when(c < nch)
                def _():
                    @pl.when(r > 0)
                    def _():
                        out_copy(c, bf).wait()
                    idx_copy(c, bf).wait()
                    gat_copy(bf).start()

            for bf in range(_NBUF):
                c = r * _NBUF + bf

                @pl.when(c < nch)
                def _():
                    gat_copy(bf).wait()
                    out_copy(c, bf).start()

                    @pl.when(c + _NBUF < nch)
                    def _():
                        idx_copy(c + _NBUF, bf).start()

            return carry

        lax.fori_loop(0, nround, round_fn, 0)
        for bf in range(min(_NBUF, nch)):
            out_copy(0, bf).wait()

    fn = pl.kernel(
        body,
        out_type=jax.ShapeDtypeStruct((b, d), jnp.float32),
        mesh=mesh,
        scratch_types=(
            [pltpu.VMEM((cb,), jnp.int32) for _ in range(_NBUF)]
            + [pltpu.VMEM((cb, d), jnp.float32) for _ in range(_NBUF)]
            + [pltpu.SemaphoreType.DMA for _ in range(3 * _NBUF)]
        ),
    )
    return fn(table, idx)


def _sconv_gemm(g, wk, bias, m_real, cin, relu=False, x=None, xw=None,
                n_out=_D):
    m_pad = g.shape[0]
    taps, _, cout = wk.shape
    width = xw if x is not None else cout
    k_full = taps * _D
    wf = jnp.zeros((taps, _D, cout), jnp.float32)
    wf = wf.at[:, :cin, :].set(wk).reshape(k_full, cout)

    def body(*refs):
        if x is not None:
            a_ref, w_ref, b_ref, x_ref, o_ref = refs
        else:
            a_ref, w_ref, b_ref, o_ref = refs
        i = pl.program_id(0)
        acc = jnp.dot(a_ref[...], w_ref[...],
                      preferred_element_type=jnp.float32)
        acc = acc + b_ref[...]
        if relu:
            acc = jnp.maximum(acc, 0.0)
        if x is not None:
            acc = x_ref[:, :xw] * acc[:, :xw] + acc[:, xw:]
        if n_out > width:
            acc = jnp.pad(acc, ((0, 0), (0, n_out - width)))
        rows = lax.broadcasted_iota(jnp.int32, (_TM, n_out), 0) + i * _TM
        o_ref[...] = jnp.where(rows < m_real, acc, 0.0)

    in_specs = [
        pl.BlockSpec((_TM, k_full), lambda i: (i, 0)),
        pl.BlockSpec((k_full, cout), lambda i: (0, 0)),
        pl.BlockSpec((1, cout), lambda i: (0, 0)),
    ]
    args = [g, wf, bias.reshape(1, cout)]
    if x is not None:
        in_specs.append(pl.BlockSpec((_TM, _D), lambda i: (i, 0)))
        args.append(x)
    return pl.pallas_call(
        body,
        grid=(m_pad // _TM,),
        in_specs=in_specs,
        out_specs=pl.BlockSpec((_TM, n_out), lambda i: (i, 0)),
        out_shape=jax.ShapeDtypeStruct((m_pad, n_out), jnp.float32),
    )(*args)


def _up_gemm(feat, wk, bias, m_real, cin):
    m_pad = feat.shape[0]
    _, _, cout = wk.shape
    w = jnp.zeros((cin, 8, _D), jnp.float32)
    w = w.at[:, :, :cout].set(wk[:, :cin, :].transpose(1, 0, 2))
    w = w.reshape(cin, 8 * _D)
    b = jnp.zeros((8, _D), jnp.float32).at[:, :cout].set(bias).reshape(1, -1)

    def body(a_ref, w_ref, b_ref, o_ref):
        i = pl.program_id(0)
        acc = jnp.dot(a_ref[:, :cin], w_ref[...],
                      preferred_element_type=jnp.float32)
        acc = jnp.maximum(acc + b_ref[...], 0.0)
        rows = lax.broadcasted_iota(jnp.int32, (_TM, 8 * _D), 0) + i * _TM
        o_ref[...] = jnp.where(rows < m_real, acc, 0.0)

    return pl.pallas_call(
        body,
        grid=(m_pad // _TM,),
        in_specs=[
            pl.BlockSpec((_TM, _D), lambda i: (i, 0)),
            pl.BlockSpec((cin, 8 * _D), lambda i: (0, 0)),
            pl.BlockSpec((1, 8 * _D), lambda i: (0, 0)),
        ],
        out_specs=pl.BlockSpec((_TM, 8 * _D), lambda i: (i, 0)),
        out_shape=jax.ShapeDtypeStruct((m_pad, 8 * _D), jnp.float32),
    )(feat, w, b)


def _neighbor_table(km, m, m_pad):
    oo = jnp.concatenate([km[k][1] for k in range(27)])
    ii = jnp.concatenate([km[k][0] for k in range(27)])
    kk = jnp.concatenate(
        [jnp.full(km[k][0].shape, k, jnp.int32) for k in range(27)]
    )
    tab = m + jnp.arange(m_pad * 27, dtype=jnp.int32) % (m_pad - m)
    return tab.at[oo * 27 + kk].min(
        ii, unique_indices=True, mode="promise_in_bounds"
    )


def _up_index(um, m_out_pad, m_in, m_in_pad):
    fi = jnp.concatenate([um[k][0] for k in range(8)])
    pk = jnp.concatenate([um[k][1] * 8 + k for k in range(8)])
    idx = (m_in + jnp.arange(m_out_pad, dtype=jnp.int32)
           % (m_in_pad - m_in)) * 8
    return idx.at[fi].min(
        pk.astype(jnp.int32), unique_indices=True, mode="promise_in_bounds"
    )


def _sconv(feat, cin, wk, bias, ntab, m, relu=False, x=None, xw=None,
           n_out=_D):
    g = _sc_gather(feat, ntab)
    a = g.reshape(-1, 27 * _D)
    return _sconv_gemm(a, wk, bias, m, cin, relu=relu, x=x, xw=xw,
                       n_out=n_out)


def _upconv(feat, cin, wk, bias, idx, m_in):
    p = _up_gemm(feat, wk, bias, m_in, cin)
    return _sc_gather(p.reshape(-1, _D), idx)


def kernel(y_feat, params, km8, km4, km2, km1, up84, up42, up21, M8, M4, M2, M1):
    p = params
    m8 = y_feat.shape[0]
    m4 = km4[13][0].shape[0]
    m2 = km2[13][0].shape[0]
    m1 = km1[13][0].shape[0]
    p8, p4, p2, p1 = (_ru(m + 512, _TM) for m in (m8, m4, m2, m1))

    n8 = _neighbor_table(km8, m8, p8)
    n4 = _neighbor_table(km4, m4, p4)
    n2 = _neighbor_table(km2, m2, p2)
    n1 = _neighbor_table(km1, m1, p1)
    i84 = _up_index(up84, p4, m8, p8)
    i42 = _up_index(up42, p2, m4, p4)
    i21 = _up_index(up21, p1, m2, p2)

    x = jnp.pad(y_feat[:, :128], ((0, p8 - m8), (0, 0)))
    q = jnp.pad(y_feat[:, 128:], ((0, p8 - m8), (0, _D - 32)))

    x = _sconv(x, 128, p['pre_w'], p['pre_b'], n8, m8, relu=True)
    q = _sconv(q, 32, p['qpre_w'], p['qpre_b'], n8, m8, relu=True)
    q = _sconv(q, 32, p['ql1_w'], p['ql1_b'], n8, m8, relu=True)
    h = _sconv(q, 32, p['qp1_w1'], p['qp1_b1'], n8, m8, relu=True)
    x = _sconv(h, 128, p['qp1_w2'], p['qp1_b2'], n8, m8, x=x, xw=128)

    q = _upconv(q, 32, p['qup1_w'], p['qup1_b'], i84, m8)
    x = _upconv(x, 128, p['up1_w'], p['up1_b'], i84, m8)

    q = _sconv(q, 32, p['ql2_w'], p['ql2_b'], n4, m4, relu=True)
    h = _sconv(q, 32, p['qp2_w1'], p['qp2_b1'], n4, m4, relu=True)
    x = _sconv(h, 128, p['qp2_w2'], p['qp2_b2'], n4, m4, x=x, xw=128)

    q = _upconv(q, 32, p['qup2_w'], p['qup2_b'], i42, m4)
    x = _upconv(x, 128, p['up2_w'], p['up2_b'], i42, m4)

    q = _sconv(q, 16, p['ql3_w'], p['ql3_b'], n2, m2)
    h = _sconv(q, 16, p['qp3_w1'], p['qp3_b1'], n2, m2, relu=True)
    x = _sconv(h, 64, p['qp3_w2'], p['qp3_b2'], n2, m2, x=x, xw=64)

    x = _upconv(x, 64, p['up3_w'], p['up3_b'], i21, m2)
    x = _sconv(x, 32, p['post_w'], p['post_b'], n1, m1, n_out=3)
    return x[:m1]


# speedup vs baseline: 10.5171x; 2.1568x over previous
import jax
import jax.numpy as jnp
from jax import lax
from jax.experimental import pallas as pl
from jax.experimental.pallas import tpu as pltpu
from jax.experimental.pallas import tpu_sc as plsc

_NC, _NS = 2, 16
_NW = _NC * _NS
_TM = 512
_D = 128


def _ru(x, m):
    return (x + m - 1) // m * m


_NBUF = 4


def _pick_chunk(b_per_w, cap):
    best = 8
    e = 8
    while e <= min(b_per_w, cap):
        if b_per_w % e == 0:
            best = e
        e += 8
    return best


def _sc_gather(table, idx):
    v, d = table.shape
    (b,) = idx.shape
    b_per_w = b // _NW
    cap = 124000 // (_NBUF * (d + 1))
    cb = _pick_chunk(b_per_w, cap)
    nch = b_per_w // cb
    nround = -(-nch // _NBUF)
    mesh = plsc.VectorSubcoreMesh(core_axis_name="c", subcore_axis_name="s")

    def body(table_hbm, idx_hbm, out_hbm, *scratch):
        idx_v = scratch[0:_NBUF]
        rows_v = scratch[_NBUF:2 * _NBUF]
        sem_i = scratch[2 * _NBUF:3 * _NBUF]
        sem_g = scratch[3 * _NBUF:4 * _NBUF]
        sem_o = scratch[4 * _NBUF:5 * _NBUF]
        wid = lax.axis_index("s") * _NC + lax.axis_index("c")
        base = wid * b_per_w

        def idx_copy(c, bf):
            return pltpu.make_async_copy(
                idx_hbm.at[pl.ds(base + c * cb, cb)], idx_v[bf], sem_i[bf])

        def out_copy(c, bf):
            return pltpu.make_async_copy(
                rows_v[bf], out_hbm.at[pl.ds(base + c * cb, cb)], sem_o[bf])

        def gat_copy(bf):
            return pltpu.make_async_copy(
                table_hbm.at[idx_v[bf]], rows_v[bf], sem_g[bf])

        for bf in range(min(_NBUF, nch)):
            idx_copy(bf, bf).start()

        def round_fn(r, carry):
            for bf in range(_NBUF):
                c = r * _NBUF + bf

                @pl.when(c < nch)
                def _():
                    @pl.when(r > 0)
                    def _():
                        out_copy(c, bf).wait()
                    idx_copy(c, bf).wait()
                    gat_copy(bf).start()

            for bf in range(_NBUF):
                c = r * _NBUF + bf

                @pl.when(c < nch)
                def _():
                    gat_copy(bf).wait()
                    out_copy(c, bf).start()

                    @pl.when(c + _NBUF < nch)
                    def _():
                        idx_copy(c + _NBUF, bf).start()

            return carry

        lax.fori_loop(0, nround, round_fn, 0)
        for bf in range(min(_NBUF, nch)):
            out_copy(0, bf).wait()

    fn = pl.kernel(
        body,
        out_type=jax.ShapeDtypeStruct((b, d), jnp.float32),
        mesh=mesh,
        scratch_types=(
            [pltpu.VMEM((cb,), jnp.int32) for _ in range(_NBUF)]
            + [pltpu.VMEM((cb, d), jnp.float32) for _ in range(_NBUF)]
            + [pltpu.SemaphoreType.DMA for _ in range(3 * _NBUF)]
        ),
    )
    return fn(table, idx)


def _sconv_gemm(g, wk, bias, m_real, cin, relu=False, x=None, xw=None,
                n_out=_D):
    m_pad = g.shape[0]
    taps, _, cout = wk.shape
    width = xw if x is not None else cout
    k_full = taps * _D
    wf = jnp.zeros((taps, _D, cout), jnp.float32)
    wf = wf.at[:, :cin, :].set(wk).reshape(k_full, cout)

    def body(*refs):
        if x is not None:
            a_ref, w_ref, b_ref, x_ref, o_ref = refs
        else:
            a_ref, w_ref, b_ref, o_ref = refs
        i = pl.program_id(0)
        acc = jnp.dot(a_ref[...], w_ref[...],
                      preferred_element_type=jnp.float32)
        acc = acc + b_ref[...]
        if relu:
            acc = jnp.maximum(acc, 0.0)
        if x is not None:
            acc = x_ref[:, :xw] * acc[:, :xw] + acc[:, xw:]
        if n_out > width:
            acc = jnp.pad(acc, ((0, 0), (0, n_out - width)))
        rows = lax.broadcasted_iota(jnp.int32, (_TM, n_out), 0) + i * _TM
        o_ref[...] = jnp.where(rows < m_real, acc, 0.0)

    in_specs = [
        pl.BlockSpec((_TM, k_full), lambda i: (i, 0)),
        pl.BlockSpec((k_full, cout), lambda i: (0, 0)),
        pl.BlockSpec((1, cout), lambda i: (0, 0)),
    ]
    args = [g, wf, bias.reshape(1, cout)]
    if x is not None:
        in_specs.append(pl.BlockSpec((_TM, _D), lambda i: (i, 0)))
        args.append(x)
    return pl.pallas_call(
        body,
        grid=(m_pad // _TM,),
        in_specs=in_specs,
        out_specs=pl.BlockSpec((_TM, n_out), lambda i: (i, 0)),
        out_shape=jax.ShapeDtypeStruct((m_pad, n_out), jnp.float32),
    )(*args)


def _up_gemm(feat, wk, bias, m_real, cin):
    m_pad = feat.shape[0]
    _, _, cout = wk.shape
    w = jnp.zeros((cin, 8, _D), jnp.float32)
    w = w.at[:, :, :cout].set(wk[:, :cin, :].transpose(1, 0, 2))
    w = w.reshape(cin, 8 * _D)
    b = jnp.zeros((8, _D), jnp.float32).at[:, :cout].set(bias).reshape(1, -1)

    def body(a_ref, w_ref, b_ref, o_ref):
        i = pl.program_id(0)
        acc = jnp.dot(a_ref[:, :cin], w_ref[...],
                      preferred_element_type=jnp.float32)
        acc = jnp.maximum(acc + b_ref[...], 0.0)
        rows = lax.broadcasted_iota(jnp.int32, (_TM, 8 * _D), 0) + i * _TM
        o_ref[...] = jnp.where(rows < m_real, acc, 0.0)

    return pl.pallas_call(
        body,
        grid=(m_pad // _TM,),
        in_specs=[
            pl.BlockSpec((_TM, _D), lambda i: (i, 0)),
            pl.BlockSpec((cin, 8 * _D), lambda i: (0, 0)),
            pl.BlockSpec((1, 8 * _D), lambda i: (0, 0)),
        ],
        out_specs=pl.BlockSpec((_TM, 8 * _D), lambda i: (i, 0)),
        out_shape=jax.ShapeDtypeStruct((m_pad, 8 * _D), jnp.float32),
    )(feat, w, b)


def _pad_updates(lin, val):
    n = lin.shape[0]
    pad = -n % 256
    if pad:
        lin = jnp.concatenate([lin, jnp.zeros((pad,), jnp.int32)])
        val = jnp.concatenate(
            [val, jnp.full((pad,), jnp.iinfo(jnp.int32).max, jnp.int32)])
    return lin, val


def _neighbor_table(km, m, m_pad):
    oo = jnp.concatenate([km[k][1] for k in range(27)])
    ii = jnp.concatenate([km[k][0] for k in range(27)])
    kk = jnp.concatenate(
        [jnp.full(km[k][0].shape, k, jnp.int32) for k in range(27)]
    )
    lin, val = _pad_updates(oo * 27 + kk, ii)
    tab = m + jnp.arange(m_pad * 27, dtype=jnp.int32) % (m_pad - m)
    return tab.at[lin].min(
        val, unique_indices=True, mode="promise_in_bounds"
    )


def _up_index(um, m_out_pad, m_in, m_in_pad):
    fi = jnp.concatenate([um[k][0] for k in range(8)])
    pk = jnp.concatenate([um[k][1] * 8 + k for k in range(8)])
    fi, pk = _pad_updates(fi, pk.astype(jnp.int32))
    idx = (m_in + jnp.arange(m_out_pad, dtype=jnp.int32)
           % (m_in_pad - m_in)) * 8
    return idx.at[fi].min(
        pk, unique_indices=True, mode="promise_in_bounds"
    )


def _sconv(feat, cin, wk, bias, ntab, m, relu=False, x=None, xw=None,
           n_out=_D):
    g = _sc_gather(feat, ntab)
    a = g.reshape(-1, 27 * _D)
    return _sconv_gemm(a, wk, bias, m, cin, relu=relu, x=x, xw=xw,
                       n_out=n_out)


def _upconv(feat, cin, wk, bias, idx, m_in):
    p = _up_gemm(feat, wk, bias, m_in, cin)
    return _sc_gather(p.reshape(-1, _D), idx)


def kernel(y_feat, params, km8, km4, km2, km1, up84, up42, up21, M8, M4, M2, M1):
    p = params
    m8 = y_feat.shape[0]
    m4 = km4[13][0].shape[0]
    m2 = km2[13][0].shape[0]
    m1 = km1[13][0].shape[0]
    p8, p4, p2, p1 = (_ru(m + 512, _TM) for m in (m8, m4, m2, m1))

    n8 = _neighbor_table(km8, m8, p8)
    n4 = _neighbor_table(km4, m4, p4)
    n2 = _neighbor_table(km2, m2, p2)
    n1 = _neighbor_table(km1, m1, p1)
    i84 = _up_index(up84, p4, m8, p8)
    i42 = _up_index(up42, p2, m4, p4)
    i21 = _up_index(up21, p1, m2, p2)

    x = jnp.pad(y_feat[:, :128], ((0, p8 - m8), (0, 0)))
    q = jnp.pad(y_feat[:, 128:], ((0, p8 - m8), (0, _D - 32)))

    x = _sconv(x, 128, p['pre_w'], p['pre_b'], n8, m8, relu=True)
    q = _sconv(q, 32, p['qpre_w'], p['qpre_b'], n8, m8, relu=True)
    q = _sconv(q, 32, p['ql1_w'], p['ql1_b'], n8, m8, relu=True)
    h = _sconv(q, 32, p['qp1_w1'], p['qp1_b1'], n8, m8, relu=True)
    x = _sconv(h, 128, p['qp1_w2'], p['qp1_b2'], n8, m8, x=x, xw=128)

    q = _upconv(q, 32, p['qup1_w'], p['qup1_b'], i84, m8)
    x = _upconv(x, 128, p['up1_w'], p['up1_b'], i84, m8)

    q = _sconv(q, 32, p['ql2_w'], p['ql2_b'], n4, m4, relu=True)
    h = _sconv(q, 32, p['qp2_w1'], p['qp2_b1'], n4, m4, relu=True)
    x = _sconv(h, 128, p['qp2_w2'], p['qp2_b2'], n4, m4, x=x, xw=128)

    q = _upconv(q, 32, p['qup2_w'], p['qup2_b'], i42, m4)
    x = _upconv(x, 128, p['up2_w'], p['up2_b'], i42, m4)

    q = _sconv(q, 16, p['ql3_w'], p['ql3_b'], n2, m2)
    h = _sconv(q, 16, p['qp3_w1'], p['qp3_b1'], n2, m2, relu=True)
    x = _sconv(h, 64, p['qp3_w2'], p['qp3_b2'], n2, m2, x=x, xw=64)

    x = _upconv(x, 64, p['up3_w'], p['up3_b'], i21, m2)
    x = _sconv(x, 32, p['post_w'], p['post_b'], n1, m1, n_out=3)
    return x[:m1]


# speedup vs baseline: 14.4107x; 1.3702x over previous
import jax
import jax.numpy as jnp
from jax import lax
from jax.experimental import pallas as pl
from jax.experimental.pallas import tpu as pltpu
from jax.experimental.pallas import tpu_sc as plsc

_NC, _NS = 2, 16
_NW = _NC * _NS
_TM = 512
_D = 128


def _ru(x, m):
    return (x + m - 1) // m * m


_NBUF = 4


def _pick_chunk(b_per_w, cap):
    best = 8
    e = 8
    while e <= min(b_per_w, cap):
        if b_per_w % e == 0:
            best = e
        e += 8
    return best


def _sc_gather(table, idx):
    v, d = table.shape
    (b,) = idx.shape
    b_per_w = b // _NW
    cap = 124000 // (_NBUF * (d + 1))
    cb = _pick_chunk(b_per_w, cap)
    nch = b_per_w // cb
    nround = -(-nch // _NBUF)
    mesh = plsc.VectorSubcoreMesh(core_axis_name="c", subcore_axis_name="s")

    def body(table_hbm, idx_hbm, out_hbm, *scratch):
        idx_v = scratch[0:_NBUF]
        rows_v = scratch[_NBUF:2 * _NBUF]
        sem_i = scratch[2 * _NBUF:3 * _NBUF]
        sem_g = scratch[3 * _NBUF:4 * _NBUF]
        sem_o = scratch[4 * _NBUF:5 * _NBUF]
        wid = lax.axis_index("s") * _NC + lax.axis_index("c")
        base = wid * b_per_w

        def idx_copy(c, bf):
            return pltpu.make_async_copy(
                idx_hbm.at[pl.ds(base + c * cb, cb)], idx_v[bf], sem_i[bf])

        def out_copy(c, bf):
            return pltpu.make_async_copy(
                rows_v[bf], out_hbm.at[pl.ds(base + c * cb, cb)], sem_o[bf])

        def gat_copy(bf):
            return pltpu.make_async_copy(
                table_hbm.at[idx_v[bf]], rows_v[bf], sem_g[bf])

        for bf in range(min(_NBUF, nch)):
            idx_copy(bf, bf).start()

        def round_fn(r, carry):
            for bf in range(_NBUF):
                c = r * _NBUF + bf

                @pl.when(c < nch)
                def _():
                    @pl.when(r > 0)
                    def _():
                        out_copy(c, bf).wait()
                    idx_copy(c, bf).wait()
                    gat_copy(bf).start()

            for bf in range(_NBUF):
                c = r * _NBUF + bf

                @pl.when(c < nch)
                def _():
                    gat_copy(bf).wait()
                    out_copy(c, bf).start()

                    @pl.when(c + _NBUF < nch)
                    def _():
                        idx_copy(c + _NBUF, bf).start()

            return carry

        lax.fori_loop(0, nround, round_fn, 0)
        for bf in range(min(_NBUF, nch)):
            out_copy(0, bf).wait()

    fn = pl.kernel(
        body,
        out_type=jax.ShapeDtypeStruct((b, d), jnp.float32),
        mesh=mesh,
        scratch_types=(
            [pltpu.VMEM((cb,), jnp.int32) for _ in range(_NBUF)]
            + [pltpu.VMEM((cb, d), jnp.float32) for _ in range(_NBUF)]
            + [pltpu.SemaphoreType.DMA for _ in range(3 * _NBUF)]
        ),
    )
    return fn(table, idx)


def _sconv_gemm(g, wk, bias, m_real, cin, relu=False, x=None, xw=None,
                n_out=_D):
    m_pad = g.shape[0] // 27
    taps, _, cout = wk.shape
    width = xw if x is not None else cout
    k_full = taps * _D
    wf = jnp.zeros((taps, _D, cout), jnp.float32)
    wf = wf.at[:, :cin, :].set(wk).reshape(k_full, cout)

    def body(*refs):
        if x is not None:
            a_ref, w_ref, b_ref, x_ref, o_ref = refs
        else:
            a_ref, w_ref, b_ref, o_ref = refs
        i = pl.program_id(0)
        a = a_ref[...].reshape(_TM, k_full)
        acc = jnp.dot(a, w_ref[...],
                      preferred_element_type=jnp.float32)
        acc = acc + b_ref[...]
        if relu:
            acc = jnp.maximum(acc, 0.0)
        if x is not None:
            acc = x_ref[:, :xw] * acc[:, :xw] + acc[:, xw:]
        if n_out > width:
            acc = jnp.pad(acc, ((0, 0), (0, n_out - width)))
        rows = lax.broadcasted_iota(jnp.int32, (_TM, n_out), 0) + i * _TM
        o_ref[...] = jnp.where(rows < m_real, acc, 0.0)

    in_specs = [
        pl.BlockSpec((_TM * 27, _D), lambda i: (i, 0)),
        pl.BlockSpec((k_full, cout), lambda i: (0, 0)),
        pl.BlockSpec((1, cout), lambda i: (0, 0)),
    ]
    args = [g, wf, bias.reshape(1, cout)]
    if x is not None:
        in_specs.append(pl.BlockSpec((_TM, _D), lambda i: (i, 0)))
        args.append(x)
    return pl.pallas_call(
        body,
        grid=(m_pad // _TM,),
        in_specs=in_specs,
        out_specs=pl.BlockSpec((_TM, n_out), lambda i: (i, 0)),
        out_shape=jax.ShapeDtypeStruct((m_pad, n_out), jnp.float32),
    )(*args)


def _up_gemm(feat, wk, bias, m_real, cin):
    m_pad = feat.shape[0]
    _, _, cout = wk.shape
    w = jnp.zeros((cin, 8, _D), jnp.float32)
    w = w.at[:, :, :cout].set(wk[:, :cin, :].transpose(1, 0, 2))
    w = w.reshape(cin, 8 * _D)
    b = jnp.zeros((8, _D), jnp.float32).at[:, :cout].set(bias).reshape(1, -1)

    def body(a_ref, w_ref, b_ref, o_ref):
        i = pl.program_id(0)
        acc = jnp.dot(a_ref[:, :cin], w_ref[...],
                      preferred_element_type=jnp.float32)
        acc = jnp.maximum(acc + b_ref[...], 0.0)
        rows = lax.broadcasted_iota(jnp.int32, (_TM, 8 * _D), 0) + i * _TM
        acc = jnp.where(rows < m_real, acc, 0.0)
        o_ref[...] = acc.reshape(_TM * 8, _D)

    return pl.pallas_call(
        body,
        grid=(m_pad // _TM,),
        in_specs=[
            pl.BlockSpec((_TM, _D), lambda i: (i, 0)),
            pl.BlockSpec((cin, 8 * _D), lambda i: (0, 0)),
            pl.BlockSpec((1, 8 * _D), lambda i: (0, 0)),
        ],
        out_specs=pl.BlockSpec((_TM * 8, _D), lambda i: (i, 0)),
        out_shape=jax.ShapeDtypeStruct((m_pad * 8, _D), jnp.float32),
    )(feat, w, b)


def _pad_updates(lin, val):
    n = lin.shape[0]
    pad = -n % 256
    if pad:
        lin = jnp.concatenate([lin, jnp.zeros((pad,), jnp.int32)])
        val = jnp.concatenate(
            [val, jnp.full((pad,), jnp.iinfo(jnp.int32).max, jnp.int32)])
    return lin, val


def _neighbor_table(km, m, m_pad):
    oo = jnp.concatenate([km[k][1] for k in range(27)])
    ii = jnp.concatenate([km[k][0] for k in range(27)])
    kk = jnp.concatenate(
        [jnp.full(km[k][0].shape, k, jnp.int32) for k in range(27)]
    )
    lin, val = _pad_updates(oo * 27 + kk, ii)
    tab = m + jnp.arange(m_pad * 27, dtype=jnp.int32) % (m_pad - m)
    return tab.at[lin].min(
        val, unique_indices=True, mode="promise_in_bounds"
    )


def _up_index(um, m_out_pad, m_in, m_in_pad):
    fi = jnp.concatenate([um[k][0] for k in range(8)])
    pk = jnp.concatenate([um[k][1] * 8 + k for k in range(8)])
    fi, pk = _pad_updates(fi, pk.astype(jnp.int32))
    idx = (m_in + jnp.arange(m_out_pad, dtype=jnp.int32)
           % (m_in_pad - m_in)) * 8
    return idx.at[fi].min(
        pk, unique_indices=True, mode="promise_in_bounds"
    )


def _sconv(feat, cin, wk, bias, ntab, m, relu=False, x=None, xw=None,
           n_out=_D):
    g = _sc_gather(feat, ntab)
    return _sconv_gemm(g, wk, bias, m, cin, relu=relu, x=x, xw=xw,
                       n_out=n_out)


def _upconv(feat, cin, wk, bias, idx, m_in):
    p = _up_gemm(feat, wk, bias, m_in, cin)
    return _sc_gather(p, idx)


def kernel(y_feat, params, km8, km4, km2, km1, up84, up42, up21, M8, M4, M2, M1):
    p = params
    m8 = y_feat.shape[0]
    m4 = km4[13][0].shape[0]
    m2 = km2[13][0].shape[0]
    m1 = km1[13][0].shape[0]
    p8, p4, p2, p1 = (_ru(m + 512, _TM) for m in (m8, m4, m2, m1))

    n8 = _neighbor_table(km8, m8, p8)
    n4 = _neighbor_table(km4, m4, p4)
    n2 = _neighbor_table(km2, m2, p2)
    n1 = _neighbor_table(km1, m1, p1)
    i84 = _up_index(up84, p4, m8, p8)
    i42 = _up_index(up42, p2, m4, p4)
    i21 = _up_index(up21, p1, m2, p2)

    x = jnp.pad(y_feat[:, :128], ((0, p8 - m8), (0, 0)))
    q = jnp.pad(y_feat[:, 128:], ((0, p8 - m8), (0, _D - 32)))

    x = _sconv(x, 128, p['pre_w'], p['pre_b'], n8, m8, relu=True)
    q = _sconv(q, 32, p['qpre_w'], p['qpre_b'], n8, m8, relu=True)
    q = _sconv(q, 32, p['ql1_w'], p['ql1_b'], n8, m8, relu=True)
    h = _sconv(q, 32, p['qp1_w1'], p['qp1_b1'], n8, m8, relu=True)
    x = _sconv(h, 128, p['qp1_w2'], p['qp1_b2'], n8, m8, x=x, xw=128)

    q = _upconv(q, 32, p['qup1_w'], p['qup1_b'], i84, m8)
    x = _upconv(x, 128, p['up1_w'], p['up1_b'], i84, m8)

    q = _sconv(q, 32, p['ql2_w'], p['ql2_b'], n4, m4, relu=True)
    h = _sconv(q, 32, p['qp2_w1'], p['qp2_b1'], n4, m4, relu=True)
    x = _sconv(h, 128, p['qp2_w2'], p['qp2_b2'], n4, m4, x=x, xw=128)

    q = _upconv(q, 32, p['qup2_w'], p['qup2_b'], i42, m4)
    x = _upconv(x, 128, p['up2_w'], p['up2_b'], i42, m4)

    q = _sconv(q, 16, p['ql3_w'], p['ql3_b'], n2, m2)
    h = _sconv(q, 16, p['qp3_w1'], p['qp3_b1'], n2, m2, relu=True)
    x = _sconv(h, 64, p['qp3_w2'], p['qp3_b2'], n2, m2, x=x, xw=64)

    x = _upconv(x, 64, p['up3_w'], p['up3_b'], i21, m2)
    x = _sconv(x, 32, p['post_w'], p['post_b'], n1, m1, n_out=3)
    return x[:m1]
